# baseline (speedup 1.0000x reference)
import functools

import jax
import jax.numpy as jnp
from jax import lax
from jax.experimental import pallas as pl
from jax.experimental.pallas import tpu as pltpu
from jax.experimental.pallas import tpu_sc as plsc

_D = 256
_S = 8192
_B = 4
_NC = 2
_NS = 16
_NW = _NC * _NS
_ROWS = _S // _NW
_SC_B = 2
_BS = 512

_mesh = plsc.VectorSubcoreMesh(core_axis_name="c", subcore_axis_name="s")


@functools.partial(
    pl.kernel,
    mesh=_mesh,
    out_type=jax.ShapeDtypeStruct((_B, _S, _D), jnp.float32),
    scratch_types=[
        pltpu.VMEM((_ROWS, _D), jnp.float32),
        pltpu.SemaphoreType.DMA,
    ],
)
def _sc_broadcast(emb_hbm, out_hbm, buf, sem):
    wid = lax.axis_index("s") * _NC + lax.axis_index("c")
    base = wid * _ROWS
    pltpu.sync_copy(emb_hbm.at[pl.ds(base, _ROWS)], buf)
    writes = [
        pltpu.async_copy(buf, out_hbm.at[b, pl.ds(base, _ROWS)], sem)
        for b in range(_SC_B)
    ]
    for w in writes:
        w.wait()


def _tc_body(buf_ref, emb_ref, out_ref):
    del buf_ref
    rows = emb_ref[...]
    out_ref[...] = jnp.broadcast_to(rows[None], (_B - _SC_B, _BS, _D))


def kernel(seq_in_embeds, embedding):
    del seq_in_embeds
    buf = _sc_broadcast(embedding)
    return pl.pallas_call(
        _tc_body,
        grid=(_S // _BS,),
        in_specs=[
            pl.BlockSpec(memory_space=pl.ANY),
            pl.BlockSpec((_BS, _D), lambda j: (j, 0)),
        ],
        out_specs=pl.BlockSpec((_B - _SC_B, _BS, _D), lambda j: (1, j, 0)),
        out_shape=jax.ShapeDtypeStruct((_B, _S, _D), jnp.float32),
        input_output_aliases={0: 0},
    )(buf, embedding)


# speedup vs baseline: 1.3251x; 1.3251x over previous
import functools

import jax
import jax.numpy as jnp
from jax import lax
from jax.experimental import pallas as pl
from jax.experimental.pallas import tpu as pltpu
from jax.experimental.pallas import tpu_sc as plsc

_D = 256
_S = 8192
_B = 4
_NC = 2
_NS = 16
_NW = _NC * _NS
_ROWS = _S // _NW
_CHUNK = 128

_mesh = plsc.VectorSubcoreMesh(core_axis_name="c", subcore_axis_name="s")


@functools.partial(
    pl.kernel,
    mesh=_mesh,
    out_type=jax.ShapeDtypeStruct((_B, _S, _D), jnp.float32),
    scratch_types=[
        pltpu.VMEM((_ROWS, _D), jnp.float32),
        pltpu.SemaphoreType.DMA,
        pltpu.SemaphoreType.DMA,
    ],
)
def _broadcast_rows(emb_hbm, out_hbm, buf, rsem, wsem):
    wid = lax.axis_index("s") * _NC + lax.axis_index("c")
    base = wid * _ROWS
    nchunks = _ROWS // _CHUNK
    reads = [
        pltpu.async_copy(
            emb_hbm.at[pl.ds(base + i * _CHUNK, _CHUNK)],
            buf.at[pl.ds(i * _CHUNK, _CHUNK)],
            rsem,
        )
        for i in range(nchunks)
    ]
    writes = []
    for i in range(nchunks):
        reads[i].wait()
        writes += [
            pltpu.async_copy(
                buf.at[pl.ds(i * _CHUNK, _CHUNK)],
                out_hbm.at[b, pl.ds(base + i * _CHUNK, _CHUNK)],
                wsem,
            )
            for b in range(_B)
        ]
    for w in writes:
        w.wait()


def kernel(seq_in_embeds, embedding):
    del seq_in_embeds
    return _broadcast_rows(embedding)
